# baseline (speedup 1.0000x reference)
import functools
import math

import jax
import jax.numpy as jnp
from jax import lax
from jax.experimental import pallas as pl
from jax.experimental.pallas import tpu as pltpu
from jax.experimental.pallas import tpu_sc as plsc

N = 10000
E = 320000
D = 128
R_MAX = 5.0
MLP_IN = 2 * D + 9
TB_W = 128

NB = 1000
NCHUNK = 4
CE = E // NCHUNK
EB = 3200
CB = CE // EB
GW = 128
GW_S = 128
NSUB = 16
N_PAD = 10240
ROWS_PER_SUB = N_PAD // NSUB
ZR = 32

_f32 = jnp.float32


def _prep_body(nf_ref, wsc_ref, wup_ref, a_ref, b_ref, ts_ref, tr_ref):
    nf = nf_ref[...]
    ns = jnp.dot(nf, wsc_ref[...], preferred_element_type=_f32)
    up = jnp.dot(nf, wup_ref[...], preferred_element_type=_f32)
    p = jnp.dot(ns, a_ref[...], preferred_element_type=_f32)
    q = jnp.dot(ns, b_ref[...], preferred_element_type=_f32)
    s1 = jnp.sum(ns, axis=1, keepdims=True)
    s2 = jnp.sum(ns * ns, axis=1, keepdims=True)
    pad = jnp.zeros((nf.shape[0], 62), _f32)
    psrow = jnp.concatenate([p, s1, s2, pad], axis=1)
    ub = jax.lax.bitcast_convert_type(up, jnp.uint32)
    ub = (ub + jnp.uint32(0x8000)) & jnp.uint32(0xFFFF0000)
    pb = jax.lax.bitcast_convert_type(psrow, jnp.uint32)
    pb = jax.lax.shift_right_logical(pb + jnp.uint32(0x8000), jnp.uint32(16))
    ts_ref[...] = ub | pb
    tr_ref[...] = jnp.concatenate([q, s1, s2, pad], axis=1)


def _prep_call(nf, wsc, wup, a, b):
    return pl.pallas_call(
        _prep_body,
        grid=(N // NB,),
        in_specs=[
            pl.BlockSpec((NB, D), lambda i: (i, 0)),
            pl.BlockSpec((D, D), lambda i: (0, 0)),
            pl.BlockSpec((D, D), lambda i: (0, 0)),
            pl.BlockSpec((D, 64), lambda i: (0, 0)),
            pl.BlockSpec((D, 64), lambda i: (0, 0)),
        ],
        out_specs=[
            pl.BlockSpec((NB, D), lambda i: (i, 0)),
            pl.BlockSpec((NB, TB_W), lambda i: (i, 0)),
        ],
        out_shape=[
            jax.ShapeDtypeStruct((N, D), jnp.uint32),
            jax.ShapeDtypeStruct((N, TB_W), _f32),
        ],
    )(nf, wsc, wup, a, b)


def _gather_call(table, idx, out_sds, chunk):
    mesh = plsc.VectorSubcoreMesh(core_axis_name="c", subcore_axis_name="s")
    off = chunk * (CE // GW)
    tail = table.shape[1:]
    blk = (GW,) + tail
    zeros = (0,) * len(tail)

    @functools.partial(pl.kernel, out_type=out_sds, mesh=mesh)
    def gather_kernel(tab_hbm, idx_hbm, out_hbm):
        def body(idx_v, out_v):
            pltpu.sync_copy(tab_hbm.at[idx_v.at[0]], out_v)

        pltpu.emit_pipeline(
            body,
            grid=(CE // GW,),
            in_specs=[pl.BlockSpec((1, GW), lambda i: (0, i + off))],
            out_specs=[pl.BlockSpec(blk, lambda i: (i,) + zeros)],
            core_axis_name=("c", "s"),
            dimension_semantics=(pltpu.PARALLEL,),
        )(idx_hbm, out_hbm)

    return gather_kernel(table, idx)


def _edge_body(tsg_ref, trg_ref, el_ref, t_ref, len_ref, ea_ref,
               c8_ref, pmat_ref, w2_ref, w3_ref, w4_ref, mji_ref):
    packed = tsg_ref[...]
    psg = jax.lax.bitcast_convert_type(
        jax.lax.shift_left(packed, jnp.uint32(16)), _f32)
    el = el_ref[...]
    tt = t_ref[...]
    ln = len_ref[...]
    ea = ea_ref[...]
    u = el * _f32(1.0 / R_MAX)
    u2 = u * u
    u5 = u2 * u2 * u
    c = (1.0 - 21.0 * u5 + 35.0 * u5 * u - 15.0 * u5 * u2)
    c = c * (el < R_MAX).astype(_f32)
    nn = lax.broadcasted_iota(jnp.int32, (8, 1), 0).astype(_f32) + 1.0
    npr = nn * _f32(math.pi / R_MAX)
    bessel = _f32(math.sqrt(2.0 / R_MAX)) * jnp.sin(npr * el) / el
    emb = bessel * jnp.exp(-(npr * npr) * tt) * c
    lc = ln * c
    instats = jnp.transpose(
        jnp.concatenate([psg[:, 64:66], trg_ref[:, 64:66]], axis=1))
    ssum = c * (instats[0:1] + instats[2:3]) \
        + jnp.sum(emb, axis=0, keepdims=True) + lc
    mean_ln = ssum * _f32(1.0 / MLP_IN)
    sumsq = (c * c) * (instats[1:2] + instats[3:4]) \
        + jnp.sum(emb * emb, axis=0, keepdims=True) + lc * lc
    var_ln = (sumsq - _f32(MLP_IN) * mean_ln * mean_ln) * _f32(1.0 / (MLP_IN - 1))
    vw0 = pmat_ref[3:4, 0:1]
    dinv = _f32(1.0 / math.sqrt(MLP_IN)) * jax.lax.rsqrt(vw0 * var_ln + 1e-5)
    zero4 = jnp.zeros((4, emb.shape[1]), _f32)
    x16 = jnp.transpose(
        jnp.concatenate([emb, lc, mean_ln, c, dinv, zero4], axis=0))
    c_t = x16[:, 10:11]
    dinv_t = x16[:, 11:12]
    xw = (c_t * (psg[:, 0:64] + trg_ref[:, 0:64])
          + jnp.dot(x16, c8_ref[...], preferred_element_type=_f32))
    h1in = xw * dinv_t + pmat_ref[2:3, :]
    h = h1in * (1.0 / (1.0 + jnp.exp(-h1in)))
    z = jnp.dot(h, w2_ref[...], preferred_element_type=_f32)
    h = z * (1.0 / (1.0 + jnp.exp(-z)))
    z = jnp.dot(h, w3_ref[...], preferred_element_type=_f32)
    h = z * (1.0 / (1.0 + jnp.exp(-z)))
    tpw = jnp.dot(h, w4_ref[...], preferred_element_type=_f32)
    upg = jax.lax.bitcast_convert_type(packed & jnp.uint32(0xFFFF0000), _f32)
    mji_ref[...] = upg * ea * tpw


def _edge_call(tsg, trg, el, t, ln, ea, c8, pmat, w2, w3, w4, chunk):
    off = chunk * CB
    return pl.pallas_call(
        _edge_body,
        grid=(CB,),
        in_specs=[
            pl.BlockSpec((EB, D), lambda i: (i, 0)),
            pl.BlockSpec((EB, TB_W), lambda i: (i, 0)),
            pl.BlockSpec((1, EB), lambda i: (0, i + off)),
            pl.BlockSpec((1, EB), lambda i: (0, i + off)),
            pl.BlockSpec((1, EB), lambda i: (0, i + off)),
            pl.BlockSpec((EB, 1), lambda i: (i + off, 0)),
            pl.BlockSpec((16, 64), lambda i: (0, 0)),
            pl.BlockSpec((8, 64), lambda i: (0, 0)),
            pl.BlockSpec((64, 64), lambda i: (0, 0)),
            pl.BlockSpec((64, 64), lambda i: (0, 0)),
            pl.BlockSpec((64, D), lambda i: (0, 0)),
        ],
        out_specs=pl.BlockSpec((EB, D), lambda i: (i, 0)),
        out_shape=jax.ShapeDtypeStruct((CE, D), _f32),
    )(tsg, trg, el, t, ln, ea, c8, pmat, w2, w3, w4)


def _scatter_call(mjis, rcv, chunks):
    mesh = plsc.VectorSubcoreMesh(core_axis_name="c", subcore_axis_name="s")

    @functools.partial(
        pl.kernel,
        out_type=jax.ShapeDtypeStruct((2, N_PAD, D), _f32),
        mesh=mesh,
        scratch_types=[
            pltpu.VMEM_SHARED((N_PAD, D), _f32),
            pltpu.VMEM((ZR, D), _f32),
        ],
    )
    def scatter_kernel(*refs):
        mji_hbms = refs[:len(mjis)]
        rcv_hbm = refs[len(mjis)]
        out_hbm = refs[len(mjis) + 1]
        shared = refs[len(mjis) + 2]
        zbuf = refs[len(mjis) + 3]
        cid = lax.axis_index("c")
        sid = lax.axis_index("s")

        @pl.loop(0, ZR)
        def _(r):
            @pl.loop(0, D, step=16)
            def _(cc):
                zbuf.at[r, pl.ds(cc, 16)][...] = jnp.zeros((16,), _f32)

        @pl.loop(0, ROWS_PER_SUB, step=ZR)
        def _(j):
            pltpu.sync_copy(zbuf, shared.at[pl.ds(sid * ROWS_PER_SUB + j, ZR)])

        plsc.subcore_barrier()

        def body(mji_v, rcv_v):
            pltpu.sync_copy(mji_v, shared.at[rcv_v.at[0]], add=True)

        for mji_hbm, chunk in zip(mji_hbms, chunks):
            off = chunk * (CE // GW_S)
            pltpu.emit_pipeline(
                body,
                grid=(CE // GW_S,),
                in_specs=[
                    pl.BlockSpec((GW_S, D), lambda i: (i, 0)),
                    pl.BlockSpec((1, GW_S), lambda i, off=off: (0, i + off)),
                ],
                out_specs=[],
                core_axis_name=("c", "s"),
                dimension_semantics=(pltpu.PARALLEL,),
            )(mji_hbm, rcv_hbm)

        plsc.subcore_barrier()

        pltpu.sync_copy(
            shared.at[pl.ds(sid * ROWS_PER_SUB, ROWS_PER_SUB)],
            out_hbm.at[cid, pl.ds(sid * ROWS_PER_SUB, ROWS_PER_SUB)],
        )

    return scatter_kernel(*mjis, rcv)


def _final_body(pa_ref, pb_ref, wout_ref, out_ref):
    m = pa_ref[0] + pa_ref[1] + pb_ref[0] + pb_ref[1]
    out_ref[...] = jnp.dot(m, wout_ref[...], preferred_element_type=_f32)


def _final_call(parts_a, parts_b, wout):
    return pl.pallas_call(
        _final_body,
        grid=(N // NB,),
        in_specs=[
            pl.BlockSpec((2, NB, D), lambda i: (0, i, 0)),
            pl.BlockSpec((2, NB, D), lambda i: (0, i, 0)),
            pl.BlockSpec((D, D), lambda i: (0, 0)),
        ],
        out_specs=pl.BlockSpec((NB, D), lambda i: (i, 0)),
        out_shape=jax.ShapeDtypeStruct((N, D), _f32),
    )(parts_a, parts_b, wout)


def kernel(node_feats, edge_attrs, edge_feats, lengths, edge_index,
           W_scalar, W_up, W1, W2, W3, W4, W_out,
           sn_weight, sn_bias, mean_weight, var_weight):
    snd = edge_index[0].astype(jnp.int32).reshape(1, E)
    rcv = edge_index[1].astype(jnp.int32).reshape(1, E)
    el = edge_feats[0].reshape(1, E)
    t = edge_feats[1].reshape(1, E)
    ln_row = lengths.reshape(1, E)
    inv_sqrt_d = 1.0 / math.sqrt(D)
    wsc = W_scalar * inv_sqrt_d
    wup = W_up * inv_sqrt_d
    w1w = sn_weight.reshape(-1, 1) * W1
    a = w1w[:D]
    b = w1w[D:2 * D]
    c8 = w1w[2 * D:2 * D + 8]
    drow = w1w[2 * D + 8]
    colsum = jnp.sum(w1w, axis=0)
    rowb = (sn_bias @ W1).reshape(-1)
    mw0 = jax.nn.softmax(mean_weight)[0]
    vw0 = jax.nn.softmax(var_weight)[0]
    pmat = jnp.zeros((8, 64), _f32)
    pmat = pmat.at[2].set(rowb * (1.0 / math.sqrt(MLP_IN)))
    pmat = pmat.at[3].set(vw0)
    w16 = jnp.zeros((16, 64), _f32)
    w16 = w16.at[0:8].set(c8)
    w16 = w16.at[8].set(drow)
    w16 = w16.at[9].set(-mw0 * colsum)
    w2 = W2 * 0.125
    w3 = W3 * 0.125
    w4 = W4 * 0.125
    wout = W_out * (inv_sqrt_d / 32.0)

    t_s, t_r = _prep_call(node_feats, wsc, wup, a, b)
    mjis = []
    gathered = []
    for k in range(NCHUNK):
        tsg = _gather_call(t_s, snd,
                           jax.ShapeDtypeStruct((CE, D), jnp.uint32), k)
        trg = _gather_call(t_r, rcv,
                           jax.ShapeDtypeStruct((CE, TB_W), _f32), k)
        gathered.append((tsg, trg))
    for k in range(NCHUNK):
        tsg, trg = gathered[k]
        mjis.append(_edge_call(tsg, trg, el, t, ln_row, edge_attrs,
                               w16, pmat, w2, w3, w4, k))
    ha = 1
    parts_a = _scatter_call(mjis[:ha], rcv, list(range(ha)))
    parts_b = _scatter_call(mjis[ha:], rcv, list(range(ha, NCHUNK)))
    out = _final_call(parts_a, parts_b, wout)
    return out[:, :, None]


# speedup vs baseline: 1.0382x; 1.0382x over previous
import functools
import math

import jax
import jax.numpy as jnp
from jax import lax
from jax.experimental import pallas as pl
from jax.experimental.pallas import tpu as pltpu
from jax.experimental.pallas import tpu_sc as plsc

N = 10000
E = 320000
D = 128
R_MAX = 5.0
MLP_IN = 2 * D + 9
TB_W = 128

NB = 1000
NCHUNK = 4
CE = E // NCHUNK
EB = 3200
CB = CE // EB
GW = 128
GW_S = 128
NSUB = 16
N_PAD = 10240
ROWS_PER_SUB = N_PAD // NSUB
ZR = 32

_f32 = jnp.float32


def _prep_body(nf_ref, wsc_ref, wup_ref, a_ref, b_ref, ts_ref, tr_ref):
    nf = nf_ref[...]
    ns = jnp.dot(nf, wsc_ref[...], preferred_element_type=_f32)
    up = jnp.dot(nf, wup_ref[...], preferred_element_type=_f32)
    p = jnp.dot(ns, a_ref[...], preferred_element_type=_f32)
    q = jnp.dot(ns, b_ref[...], preferred_element_type=_f32)
    s1 = jnp.sum(ns, axis=1, keepdims=True)
    s2 = jnp.sum(ns * ns, axis=1, keepdims=True)
    pad = jnp.zeros((nf.shape[0], 62), _f32)
    psrow = jnp.concatenate([p, s1, s2, pad], axis=1)
    ub = jax.lax.bitcast_convert_type(up, jnp.uint32)
    ub = (ub + jnp.uint32(0x8000)) & jnp.uint32(0xFFFF0000)
    pb = jax.lax.bitcast_convert_type(psrow, jnp.uint32)
    pb = jax.lax.shift_right_logical(pb + jnp.uint32(0x8000), jnp.uint32(16))
    ts_ref[...] = ub | pb
    tr_ref[...] = jnp.concatenate([q, s1, s2, pad], axis=1)


def _prep_call(nf, wsc, wup, a, b):
    return pl.pallas_call(
        _prep_body,
        grid=(N // NB,),
        in_specs=[
            pl.BlockSpec((NB, D), lambda i: (i, 0)),
            pl.BlockSpec((D, D), lambda i: (0, 0)),
            pl.BlockSpec((D, D), lambda i: (0, 0)),
            pl.BlockSpec((D, 64), lambda i: (0, 0)),
            pl.BlockSpec((D, 64), lambda i: (0, 0)),
        ],
        out_specs=[
            pl.BlockSpec((NB, D), lambda i: (i, 0)),
            pl.BlockSpec((NB, TB_W), lambda i: (i, 0)),
        ],
        out_shape=[
            jax.ShapeDtypeStruct((N, D), jnp.uint32),
            jax.ShapeDtypeStruct((N, TB_W), _f32),
        ],
    )(nf, wsc, wup, a, b)


def _gather_call(table, idx, out_sds, chunk):
    mesh = plsc.VectorSubcoreMesh(core_axis_name="c", subcore_axis_name="s")
    off = chunk * (CE // GW)
    tail = table.shape[1:]
    blk = (GW,) + tail
    zeros = (0,) * len(tail)

    @functools.partial(pl.kernel, out_type=out_sds, mesh=mesh)
    def gather_kernel(tab_hbm, idx_hbm, out_hbm):
        def body(idx_v, out_v):
            pltpu.sync_copy(tab_hbm.at[idx_v.at[0]], out_v)

        pltpu.emit_pipeline(
            body,
            grid=(CE // GW,),
            in_specs=[pl.BlockSpec((1, GW), lambda i: (0, i + off))],
            out_specs=[pl.BlockSpec(blk, lambda i: (i,) + zeros)],
            core_axis_name=("c", "s"),
            dimension_semantics=(pltpu.PARALLEL,),
        )(idx_hbm, out_hbm)

    return gather_kernel(table, idx)


def _edge_body(tsg_ref, trg_ref, el_ref, t_ref, len_ref, ea_ref,
               c8_ref, pmat_ref, w2_ref, w3_ref, w4_ref, mji_ref):
    packed = tsg_ref[...]
    psg = jax.lax.bitcast_convert_type(
        jax.lax.shift_left(packed, jnp.uint32(16)), _f32)
    el = el_ref[...]
    tt = t_ref[...]
    ln = len_ref[...]
    ea = ea_ref[...]
    u = el * _f32(1.0 / R_MAX)
    u2 = u * u
    u5 = u2 * u2 * u
    c = (1.0 - 21.0 * u5 + 35.0 * u5 * u - 15.0 * u5 * u2)
    c = c * (el < R_MAX).astype(_f32)
    nn = lax.broadcasted_iota(jnp.int32, (8, 1), 0).astype(_f32) + 1.0
    npr = nn * _f32(math.pi / R_MAX)
    bessel = _f32(math.sqrt(2.0 / R_MAX)) * jnp.sin(npr * el) / el
    emb = bessel * jnp.exp(-(npr * npr) * tt) * c
    lc = ln * c
    instats = jnp.transpose(
        jnp.concatenate([psg[:, 64:66], trg_ref[:, 64:66]], axis=1))
    ssum = c * (instats[0:1] + instats[2:3]) \
        + jnp.sum(emb, axis=0, keepdims=True) + lc
    mean_ln = ssum * _f32(1.0 / MLP_IN)
    sumsq = (c * c) * (instats[1:2] + instats[3:4]) \
        + jnp.sum(emb * emb, axis=0, keepdims=True) + lc * lc
    var_ln = (sumsq - _f32(MLP_IN) * mean_ln * mean_ln) * _f32(1.0 / (MLP_IN - 1))
    vw0 = pmat_ref[3:4, 0:1]
    dinv = _f32(1.0 / math.sqrt(MLP_IN)) * jax.lax.rsqrt(vw0 * var_ln + 1e-5)
    zero4 = jnp.zeros((4, emb.shape[1]), _f32)
    x16 = jnp.transpose(
        jnp.concatenate([emb, lc, mean_ln, c, dinv, zero4], axis=0))
    c_t = x16[:, 10:11]
    dinv_t = x16[:, 11:12]
    xw = (c_t * (psg[:, 0:64] + trg_ref[:, 0:64])
          + jnp.dot(x16, c8_ref[...], preferred_element_type=_f32))
    h1in = xw * dinv_t + pmat_ref[2:3, :]
    h = h1in * (1.0 / (1.0 + jnp.exp(-h1in)))
    z = jnp.dot(h, w2_ref[...], preferred_element_type=_f32)
    h = z * (1.0 / (1.0 + jnp.exp(-z)))
    z = jnp.dot(h, w3_ref[...], preferred_element_type=_f32)
    h = z * (1.0 / (1.0 + jnp.exp(-z)))
    tpw = jnp.dot(h, w4_ref[...], preferred_element_type=_f32)
    upg = jax.lax.bitcast_convert_type(packed & jnp.uint32(0xFFFF0000), _f32)
    mji_ref[...] = upg * ea * tpw


def _edge_call(tsg, trg, el, t, ln, ea, c8, pmat, w2, w3, w4, chunk):
    off = chunk * CB
    return pl.pallas_call(
        _edge_body,
        grid=(CB,),
        in_specs=[
            pl.BlockSpec((EB, D), lambda i: (i, 0)),
            pl.BlockSpec((EB, TB_W), lambda i: (i, 0)),
            pl.BlockSpec((1, EB), lambda i: (0, i + off)),
            pl.BlockSpec((1, EB), lambda i: (0, i + off)),
            pl.BlockSpec((1, EB), lambda i: (0, i + off)),
            pl.BlockSpec((EB, 1), lambda i: (i + off, 0)),
            pl.BlockSpec((16, 64), lambda i: (0, 0)),
            pl.BlockSpec((8, 64), lambda i: (0, 0)),
            pl.BlockSpec((64, 64), lambda i: (0, 0)),
            pl.BlockSpec((64, 64), lambda i: (0, 0)),
            pl.BlockSpec((64, D), lambda i: (0, 0)),
        ],
        out_specs=pl.BlockSpec((EB, D), lambda i: (i, 0)),
        out_shape=jax.ShapeDtypeStruct((CE, D), _f32),
    )(tsg, trg, el, t, ln, ea, c8, pmat, w2, w3, w4)


def _scatter_call(mjis, rcv, chunks):
    mesh = plsc.VectorSubcoreMesh(core_axis_name="c", subcore_axis_name="s")

    @functools.partial(
        pl.kernel,
        out_type=jax.ShapeDtypeStruct((2, N_PAD, D), _f32),
        mesh=mesh,
        scratch_types=[
            pltpu.VMEM_SHARED((N_PAD, D), _f32),
            pltpu.VMEM((ZR, D), _f32),
        ],
    )
    def scatter_kernel(*refs):
        mji_hbms = refs[:len(mjis)]
        rcv_hbm = refs[len(mjis)]
        out_hbm = refs[len(mjis) + 1]
        shared = refs[len(mjis) + 2]
        zbuf = refs[len(mjis) + 3]
        cid = lax.axis_index("c")
        sid = lax.axis_index("s")

        @pl.loop(0, ZR)
        def _(r):
            @pl.loop(0, D, step=16)
            def _(cc):
                zbuf.at[r, pl.ds(cc, 16)][...] = jnp.zeros((16,), _f32)

        @pl.loop(0, ROWS_PER_SUB, step=ZR)
        def _(j):
            pltpu.sync_copy(zbuf, shared.at[pl.ds(sid * ROWS_PER_SUB + j, ZR)])

        plsc.subcore_barrier()

        def body(mji_v, rcv_v):
            pltpu.sync_copy(mji_v, shared.at[rcv_v.at[0]], add=True)

        for mji_hbm, chunk in zip(mji_hbms, chunks):
            off = chunk * (CE // GW_S)
            pltpu.emit_pipeline(
                body,
                grid=(CE // GW_S,),
                in_specs=[
                    pl.BlockSpec((GW_S, D), lambda i: (i, 0)),
                    pl.BlockSpec((1, GW_S), lambda i, off=off: (0, i + off)),
                ],
                out_specs=[],
                core_axis_name=("c", "s"),
                dimension_semantics=(pltpu.PARALLEL,),
            )(mji_hbm, rcv_hbm)

        plsc.subcore_barrier()

        pltpu.sync_copy(
            shared.at[pl.ds(sid * ROWS_PER_SUB, ROWS_PER_SUB)],
            out_hbm.at[cid, pl.ds(sid * ROWS_PER_SUB, ROWS_PER_SUB)],
        )

    return scatter_kernel(*mjis, rcv)


def _final_body(pa_ref, pb_ref, wout_ref, out_ref):
    m = pa_ref[0] + pa_ref[1] + pb_ref[0] + pb_ref[1]
    out_ref[...] = jnp.dot(m, wout_ref[...], preferred_element_type=_f32)


def _final_call(parts_a, parts_b, wout):
    return pl.pallas_call(
        _final_body,
        grid=(N // NB,),
        in_specs=[
            pl.BlockSpec((2, NB, D), lambda i: (0, i, 0)),
            pl.BlockSpec((2, NB, D), lambda i: (0, i, 0)),
            pl.BlockSpec((D, D), lambda i: (0, 0)),
        ],
        out_specs=pl.BlockSpec((NB, D), lambda i: (i, 0)),
        out_shape=jax.ShapeDtypeStruct((N, D), _f32),
    )(parts_a, parts_b, wout)


def kernel(node_feats, edge_attrs, edge_feats, lengths, edge_index,
           W_scalar, W_up, W1, W2, W3, W4, W_out,
           sn_weight, sn_bias, mean_weight, var_weight):
    snd = edge_index[0].astype(jnp.int32).reshape(1, E)
    rcv = edge_index[1].astype(jnp.int32).reshape(1, E)
    el = edge_feats[0].reshape(1, E)
    t = edge_feats[1].reshape(1, E)
    ln_row = lengths.reshape(1, E)
    inv_sqrt_d = 1.0 / math.sqrt(D)
    wsc = W_scalar * inv_sqrt_d
    wup = W_up * inv_sqrt_d
    w1w = sn_weight.reshape(-1, 1) * W1
    a = w1w[:D]
    b = w1w[D:2 * D]
    c8 = w1w[2 * D:2 * D + 8]
    drow = w1w[2 * D + 8]
    colsum = jnp.sum(w1w, axis=0)
    rowb = (sn_bias @ W1).reshape(-1)
    mw0 = jax.nn.softmax(mean_weight)[0]
    vw0 = jax.nn.softmax(var_weight)[0]
    pmat = jnp.zeros((8, 64), _f32)
    pmat = pmat.at[2].set(rowb * (1.0 / math.sqrt(MLP_IN)))
    pmat = pmat.at[3].set(vw0)
    w16 = jnp.zeros((16, 64), _f32)
    w16 = w16.at[0:8].set(c8)
    w16 = w16.at[8].set(drow)
    w16 = w16.at[9].set(-mw0 * colsum)
    w2 = W2 * 0.125
    w3 = W3 * 0.125
    w4 = W4 * 0.125
    wout = W_out * (inv_sqrt_d / 32.0)

    t_s, t_r = _prep_call(node_feats, wsc, wup, a, b)
    mjis = []
    gathered = []
    for k in range(NCHUNK):
        tsg = _gather_call(t_s, snd,
                           jax.ShapeDtypeStruct((CE, D), jnp.uint32), k)
        trg = _gather_call(t_r, rcv,
                           jax.ShapeDtypeStruct((CE, TB_W), _f32), k)
        gathered.append((tsg, trg))
    for k in range(NCHUNK):
        tsg, trg = gathered[k]
        mjis.append(_edge_call(tsg, trg, el, t, ln_row, edge_attrs,
                               w16, pmat, w2, w3, w4, k))
    ha = NCHUNK // 2
    parts_a = _scatter_call(mjis[:ha], rcv, list(range(ha)))
    parts_b = _scatter_call(mjis[ha:], rcv, list(range(ha, NCHUNK)))
    out = _final_call(parts_a, parts_b, wout)
    return out[:, :, None]
